# baseline (speedup 1.0000x reference)
import functools

import jax
import jax.numpy as jnp
from jax import lax
from jax.experimental import pallas as pl
from jax.experimental.pallas import tpu as pltpu
from jax.experimental.pallas import tpu_sc as plsc

MAXLEN = 200
EMBED = 64
BATCH = 1024
OUTMIN = 128

NUM_CORES = 2
NUM_SUBCORES = 16
NUM_WORKERS = NUM_CORES * NUM_SUBCORES
SEQ_PER_W = BATCH // NUM_WORKERS
LANES = 16
NJ = EMBED // LANES

GRP = 2
GLEN = GRP * MAXLEN
NGRP = SEQ_PER_W // GRP
NBUF = 4
LEAD = 2
LAG = 2


def _make_kernel():
    mesh = plsc.VectorSubcoreMesh(core_axis_name="c", subcore_axis_name="s")

    @functools.partial(
        pl.kernel,
        mesh=mesh,
        out_type=jax.ShapeDtypeStruct((BATCH, MAXLEN, OUTMIN), jnp.float32),
        scratch_types=[
            [pltpu.VMEM((GLEN,), jnp.int32)] * NGRP,
            pltpu.VMEM((MAXLEN, EMBED), jnp.float32),
            [pltpu.VMEM((GLEN, EMBED), jnp.float32)] * NBUF,
            [pltpu.SemaphoreType.DMA] * NBUF,
            [pltpu.SemaphoreType.DMA] * NBUF,
        ],
        compiler_params=pltpu.CompilerParams(use_tc_tiling_on_sc=False),
    )
    def emb_kernel(x_hbm, tok_hbm, pos_hbm, out_hbm, idx_v, pos_v, bufs,
                   gsems, ssems):
        wid = lax.axis_index("s") * NUM_CORES + lax.axis_index("c")
        seq0 = wid * SEQ_PER_W
        for t in range(NGRP):
            for k in range(GRP):
                pltpu.async_copy(x_hbm.at[seq0 + t * GRP + k],
                                 idx_v[t].at[pl.ds(k * MAXLEN, MAXLEN)],
                                 gsems[0])
        for t in range(NGRP):
            for k in range(GRP):
                pltpu.make_async_copy(
                    x_hbm.at[seq0 + t * GRP + k],
                    idx_v[t].at[pl.ds(k * MAXLEN, MAXLEN)], gsems[0]).wait()
        pltpu.sync_copy(pos_hbm, pos_v)

        def fire_gather(t):
            b = t % NBUF
            pltpu.async_copy(tok_hbm.at[idx_v[t]], bufs[b], gsems[b])

        def add_group(t):
            buf = bufs[t % NBUF]

            def body(i, c):
                p = i * 2
                for dp in range(2):
                    pos_regs = [pos_v[p + dp, pl.ds(j * LANES, LANES)]
                                for j in range(NJ)]
                    for k in range(GRP):
                        for j in range(NJ):
                            sl = pl.ds(j * LANES, LANES)
                            r = k * MAXLEN + p + dp
                            buf[r, sl] = buf[r, sl] + pos_regs[j]
                return c

            lax.fori_loop(0, MAXLEN // 2, body, 0)

        def fire_store(t):
            b = t % NBUF
            for k in range(GRP):
                pltpu.async_copy(
                    bufs[b].at[pl.ds(k * MAXLEN, MAXLEN)],
                    out_hbm.at[seq0 + t * GRP + k, :, pl.ds(0, EMBED)],
                    ssems[b])

        def drain_store(t):
            b = t % NBUF
            for k in range(GRP):
                pltpu.make_async_copy(
                    bufs[b].at[pl.ds(k * MAXLEN, MAXLEN)],
                    out_hbm.at[seq0 + t * GRP + k, :, pl.ds(0, EMBED)],
                    ssems[b]).wait()

        for t in range(LEAD):
            fire_gather(t)
        for t in range(NGRP):
            b = t % NBUF
            pltpu.make_async_copy(
                tok_hbm.at[idx_v[t]], bufs[b], gsems[b]).wait()
            add_group(t)
            fire_store(t)
            if t >= LAG:
                drain_store(t - LAG)
            if t + LEAD < NGRP:
                fire_gather(t + LEAD)
        for t in range(NGRP - LAG, NGRP):
            drain_store(t)

    return emb_kernel


_emb = _make_kernel()


def kernel(x, token_table, pos_table):
    out = _emb(x.astype(jnp.int32), token_table, pos_table)
    return out[:, :, :EMBED]
